# baseline (speedup 1.0000x reference)
import functools

import jax
import jax.numpy as jnp
from jax import lax
from jax.experimental import pallas as pl
from jax.experimental.pallas import tpu as pltpu
from jax.experimental.pallas import tpu_sc as plsc

B = 1024
N_NUM = 13
N_CAT = 26
L = 5
VH = 30522
VV = 100000
D = 768
H = 128

NC = 2
NS = 16
NW = NC * NS

CS = 64
SEG_G = 16
GPC = CS // SEG_G
R = CS * L
SV = B * N_CAT
SE = 2048
SH = SV + SE
NV = SV // NW
NH = SH // NW
BLK = 2400
VPV = 100800
VPH = 31200
NSPREAD = 512


def _table_proj_body(tab_ref, wwt_ref, s2_ref, bwt_ref, o_ref, *, v_rows):
    pid = pl.program_id(0)
    x = tab_ref[...]
    mu = jnp.mean(x, axis=1, keepdims=True)
    msq = jnp.mean(x * x, axis=1, keepdims=True)
    rstd = lax.rsqrt(jnp.maximum(msq - mu * mu, 0.0) + 1e-5)
    p = jnp.dot(x, wwt_ref[...], preferred_element_type=jnp.float32)
    z = (p - mu * s2_ref[...]) * rstd + bwt_ref[...]
    rows = pid * BLK + lax.broadcasted_iota(jnp.int32, (BLK, 1), 0)
    o_ref[...] = jnp.where(rows < v_rows, z, 0.0)


def _table_proj(tab, w, b, wt, v_rows, vp_rows):
    nb = vp_rows // BLK
    in_cap = (v_rows + BLK - 1) // BLK - 1
    wwt = w[:, None] * wt
    s2 = jnp.sum(wwt, axis=0)[None]
    bwt = (b @ wt)[None]
    f = pl.pallas_call(
        functools.partial(_table_proj_body, v_rows=v_rows),
        grid=(nb,),
        in_specs=[
            pl.BlockSpec((BLK, D), lambda i: (jnp.minimum(i, in_cap), 0)),
            pl.BlockSpec((D, H), lambda i: (0, 0)),
            pl.BlockSpec((1, H), lambda i: (0, 0)),
            pl.BlockSpec((1, H), lambda i: (0, 0)),
        ],
        out_specs=pl.BlockSpec((BLK, H), lambda i: (i, 0)),
        out_shape=jax.ShapeDtypeStruct((vp_rows, H), jnp.float32),
    )
    return f(tab, wwt, s2, bwt)


def _sc_pool_side_body(n_per_w, v_rows, has_extras, g_hbm, id_hbm, m_hbm,
                       *rest):
    if has_extras:
        eid_hbm, o_hbm, ids_v, msk_v, rows_v, acc_v, g0, g1, s0, s1 = rest
    else:
        o_hbm, ids_v, msk_v, rows_v, acc_v, g0, g1, s0, s1 = rest
    wid = lax.axis_index("s") * NC + lax.axis_index("c")
    base = wid * n_per_w
    nw = n_per_w * L
    nchunks = n_per_w // CS
    pltpu.sync_copy(id_hbm.at[pl.ds(base * L, nw)], ids_v)
    pltpu.sync_copy(m_hbm.at[pl.ds(base * L, nw)], msk_v)

    def mask_ids(j, _):
        sl = pl.ds(j * 16, 16)
        gpos = base * L + j * 16 + lax.iota(jnp.int32, 16)
        redirect = v_rows + lax.rem(gpos, NSPREAD)
        ids_v[sl] = jnp.where(msk_v[sl] != 0, ids_v[sl], redirect)
        return 0

    lax.fori_loop(0, nw // 16, mask_ids, 0)

    def gather_descs(ci, buf, sem):
        return [
            pltpu.make_async_copy(
                g_hbm.at[ids_v.at[pl.ds(ci * R + gi * SEG_G * L, SEG_G * L)]],
                rows_v.at[buf, pl.ds(gi * SEG_G * L, SEG_G * L)], sem)
            for gi in range(GPC)
        ]

    def start(descs):
        for dsc in descs:
            dsc.start()

    def wait(descs):
        for dsc in descs:
            dsc.wait()

    def scatter_desc(ci, buf, sem):
        return pltpu.make_async_copy(
            acc_v.at[buf], o_hbm.at[pl.ds(base + ci * CS, CS)],
            s0 if sem == 0 else s1)

    def compute(buf):
        def seg(si, _):
            r0 = si * L
            for dc in range(H // 16):
                sl = pl.ds(dc * 16, 16)
                acc = rows_v[buf, r0, sl] + rows_v[buf, r0 + 1, sl]
                acc = acc + rows_v[buf, r0 + 2, sl]
                acc = acc + rows_v[buf, r0 + 3, sl]
                acc = acc + rows_v[buf, r0 + 4, sl]
                acc_v[buf, si, sl] = acc
            return 0

        lax.fori_loop(0, CS, seg, 0)

    start(gather_descs(0, 0, g0))

    def pair(k, _):
        ci0 = 2 * k
        ci1 = ci0 + 1
        start(gather_descs(ci1, 1, g1))
        wait(gather_descs(ci0, 0, g0))

        @pl.when(ci0 >= 2)
        def _():
            scatter_desc(ci0 - 2, 0, 0).wait()

        compute(0)
        scatter_desc(ci0, 0, 0).start()

        @pl.when(ci1 + 1 < nchunks)
        def _():
            start(gather_descs(ci1 + 1, 0, g0))

        wait(gather_descs(ci1, 1, g1))

        @pl.when(ci1 >= 2)
        def _():
            scatter_desc(ci1 - 2, 1, 1).wait()

        compute(1)
        scatter_desc(ci1, 1, 1).start()
        return 0

    lax.fori_loop(0, nchunks // 2, pair, 0)

    if nchunks % 2:
        ci = nchunks - 1
        wait(gather_descs(ci, 0, g0))
        scatter_desc(ci - 2, 0, 0).wait()
        compute(0)
        scatter_desc(ci, 0, 0).start()

    scatter_desc(nchunks - 1, 0, 0).wait()
    scatter_desc(nchunks - 2, 1, 1).wait()

    if has_extras:
        pltpu.sync_copy(eid_hbm.at[pl.ds(wid * R, R)],
                        ids_v.at[pl.ds(0, R)])
        descs = [
            pltpu.make_async_copy(
                g_hbm.at[ids_v.at[pl.ds(gi * SEG_G * L, SEG_G * L)]],
                rows_v.at[0, pl.ds(gi * SEG_G * L, SEG_G * L)], g0)
            for gi in range(GPC)
        ]
        start(descs)
        wait(descs)
        compute(0)
        pltpu.sync_copy(acc_v.at[0],
                        o_hbm.at[pl.ds(NW * n_per_w + wid * CS, CS)])


@functools.cache
def _sc_pool_side(n_per_w, v_rows, n_extra):
    return pl.kernel(
        functools.partial(_sc_pool_side_body, n_per_w, v_rows, n_extra > 0),
        mesh=plsc.VectorSubcoreMesh(core_axis_name="c", subcore_axis_name="s"),
        out_type=jax.ShapeDtypeStruct((NW * n_per_w + n_extra, H),
                                      jnp.float32),
        scratch_types=[
            pltpu.VMEM((n_per_w * L,), jnp.int32),
            pltpu.VMEM((n_per_w * L,), jnp.int32),
            pltpu.VMEM((2, R, H), jnp.float32),
            pltpu.VMEM((2, CS, H), jnp.float32),
            pltpu.SemaphoreType.DMA,
            pltpu.SemaphoreType.DMA,
            pltpu.SemaphoreType.DMA,
            pltpu.SemaphoreType.DMA,
        ],
        compiler_params=pltpu.CompilerParams(use_tc_tiling_on_sc=False),
    )


def _assemble_body(xnum_ref, vsum_ref, hsum_ref, xm_ref, nm_ref, cm_ref,
                   ex_ref, nbias_ref, wt_ref, emb_ref, bert_ref):
    eps = 1e-12
    bb = xnum_ref.shape[0]
    ex = ex_ref[...]
    nden = jnp.sum(nm_ref[...].astype(jnp.float32), axis=1, keepdims=True) + eps
    ncp_avg = ex[N_CAT:N_CAT + N_NUM] / nden
    cden = jnp.sum(cm_ref[...].astype(jnp.float32), axis=1, keepdims=True) + eps
    colp_avg = ex[:N_CAT] / cden
    bias_p = jnp.dot(nbias_ref[...], wt_ref[...],
                     preferred_element_type=jnp.float32)
    rden = 1.0 / (jnp.sum(xm_ref[...].astype(jnp.float32), axis=2) + eps)
    vsum = vsum_ref[...].reshape(bb, N_CAT, H)
    hsum = hsum_ref[...].reshape(bb, N_CAT, H)
    val_avg = vsum * rden[:, :, None]
    hdr_avg = hsum * rden[:, :, None]
    num_part = xnum_ref[...][:, :, None] * ncp_avg[None] + bias_p[None]
    cat_part = (colp_avg[None] + val_avg) * 0.5
    emb_ref[...] = jnp.concatenate([num_part, cat_part], axis=1)
    bert_ref[...] = hdr_avg


def _assemble(xnum, vsum, hsum_all, xm, nm, cm, nbias, wt):
    BB = 128
    nb = B // BB
    f = pl.pallas_call(
        _assemble_body,
        grid=(nb,),
        in_specs=[
            pl.BlockSpec((BB, N_NUM), lambda i: (i, 0)),
            pl.BlockSpec((BB * N_CAT, H), lambda i: (i, 0)),
            pl.BlockSpec((BB * N_CAT, H), lambda i: (i, 0)),
            pl.BlockSpec((BB, N_CAT, L), lambda i: (i, 0, 0)),
            pl.BlockSpec((N_NUM, L), lambda i: (0, 0)),
            pl.BlockSpec((N_CAT, L), lambda i: (0, 0)),
            pl.BlockSpec((64, H), lambda i: (SV // 64, 0)),
            pl.BlockSpec((1, D), lambda i: (0, 0)),
            pl.BlockSpec((D, H), lambda i: (0, 0)),
        ],
        out_specs=[
            pl.BlockSpec((BB, N_NUM + N_CAT, H), lambda i: (i, 0, 0)),
            pl.BlockSpec((BB, N_CAT, H), lambda i: (i, 0, 0)),
        ],
        out_shape=[
            jax.ShapeDtypeStruct((B, N_NUM + N_CAT, H), jnp.float32),
            jax.ShapeDtypeStruct((B, N_CAT, H), jnp.float32),
        ],
    )
    return f(xnum, vsum, hsum_all, xm, nm, cm, hsum_all, nbias, wt)


def kernel(x_num, num_col_input_ids, num_att_mask, x_cat_input_ids,
           x_cat_att_mask, col_cat_input_ids, col_cat_att_mask, header_table,
           value_table, norm_header_w, norm_header_b, norm_value_w,
           norm_value_b, num_bias, align_W):
    wt = align_W.T
    gv = _table_proj(value_table, norm_value_w, norm_value_b, wt, VV, VPV)
    gh = _table_proj(header_table, norm_header_w, norm_header_b, wt, VH, VPH)

    main_ids = x_cat_input_ids.reshape(-1)
    main_msk = x_cat_att_mask.reshape(-1)
    extra_ids = jnp.concatenate(
        [col_cat_input_ids.reshape(-1), num_col_input_ids.reshape(-1),
         jnp.zeros(((SE - N_CAT - N_NUM) * L,), jnp.int32)])
    extra_msk = jnp.concatenate(
        [col_cat_att_mask.reshape(-1), num_att_mask.reshape(-1),
         jnp.zeros(((SE - N_CAT - N_NUM) * L,), jnp.int32)])
    epos = jnp.arange(SE * L, dtype=jnp.int32)
    extra_idsm = jnp.where(extra_msk != 0, extra_ids, VH + epos % NSPREAD)

    vsum = _sc_pool_side(NV, VV, 0)(gv, main_ids, main_msk)
    hsum_all = _sc_pool_side(NV, VH, SE)(gh, main_ids, main_msk, extra_idsm)

    emb, bert = _assemble(x_num, vsum, hsum_all, x_cat_att_mask, num_att_mask,
                          col_cat_att_mask, num_bias.reshape(1, D), wt)
    return emb, bert


# speedup vs baseline: 1.0566x; 1.0566x over previous
import functools

import jax
import jax.numpy as jnp
from jax import lax
from jax.experimental import pallas as pl
from jax.experimental.pallas import tpu as pltpu
from jax.experimental.pallas import tpu_sc as plsc

B = 1024
N_NUM = 13
N_CAT = 26
L = 5
VH = 30522
VV = 100000
D = 768
H = 128

NC = 2
NS = 16
NW = NC * NS

CS = 64
SEG_G = 16
GPC = CS // SEG_G
R = CS * L
SV = B * N_CAT
SE = 2048
SH = SV + SE
NV = SV // NW
NH = SH // NW
BLK = 2400
VPV = 100800
VPH = 31200
NSPREAD = 512


def _table_proj_body(tab_ref, wwt_ref, s2_ref, bwt_ref, o_ref, *, v_rows):
    pid = pl.program_id(0)
    x = tab_ref[...]
    mu = jnp.mean(x, axis=1, keepdims=True)
    msq = jnp.mean(x * x, axis=1, keepdims=True)
    rstd = lax.rsqrt(jnp.maximum(msq - mu * mu, 0.0) + 1e-5)
    p = jnp.dot(x, wwt_ref[...], preferred_element_type=jnp.float32)
    z = (p - mu * s2_ref[...]) * rstd + bwt_ref[...]
    rows = pid * BLK + lax.broadcasted_iota(jnp.int32, (BLK, 1), 0)
    o_ref[...] = jnp.where(rows < v_rows, z, 0.0)


def _table_proj(tab, w, b, wt, v_rows, vp_rows):
    nb = vp_rows // BLK
    in_cap = (v_rows + BLK - 1) // BLK - 1
    wwt = w[:, None] * wt
    s2 = jnp.sum(wwt, axis=0)[None]
    bwt = (b @ wt)[None]
    f = pl.pallas_call(
        functools.partial(_table_proj_body, v_rows=v_rows),
        grid=(nb,),
        in_specs=[
            pl.BlockSpec((BLK, D), lambda i: (jnp.minimum(i, in_cap), 0)),
            pl.BlockSpec((D, H), lambda i: (0, 0)),
            pl.BlockSpec((1, H), lambda i: (0, 0)),
            pl.BlockSpec((1, H), lambda i: (0, 0)),
        ],
        out_specs=pl.BlockSpec((BLK, H), lambda i: (i, 0)),
        out_shape=jax.ShapeDtypeStruct((vp_rows, H), jnp.float32),
    )
    return f(tab, wwt, s2, bwt)


def _sc_pool_side_body(n_per_w, has_extras, g_hbm, id_hbm, *rest):
    if has_extras:
        eid_hbm, o_hbm, ids_v, rows_v, acc_v, g0, g1, s0, s1 = rest
    else:
        o_hbm, ids_v, rows_v, acc_v, g0, g1, s0, s1 = rest
    wid = lax.axis_index("s") * NC + lax.axis_index("c")
    base = wid * n_per_w
    nw = n_per_w * L
    nchunks = n_per_w // CS
    pltpu.sync_copy(id_hbm.at[pl.ds(base * L, nw)], ids_v)

    def gather_descs(ci, buf, sem):
        return [
            pltpu.make_async_copy(
                g_hbm.at[ids_v.at[pl.ds(ci * R + gi * SEG_G * L, SEG_G * L)]],
                rows_v.at[buf, pl.ds(gi * SEG_G * L, SEG_G * L)], sem)
            for gi in range(GPC)
        ]

    def start(descs):
        for dsc in descs:
            dsc.start()

    def wait(descs):
        for dsc in descs:
            dsc.wait()

    def scatter_desc(ci, buf, sem):
        return pltpu.make_async_copy(
            acc_v.at[buf], o_hbm.at[pl.ds(base + ci * CS, CS)],
            s0 if sem == 0 else s1)

    def compute(buf):
        def seg(si, _):
            r0 = si * L
            for dc in range(H // 16):
                sl = pl.ds(dc * 16, 16)
                acc = rows_v[buf, r0, sl] + rows_v[buf, r0 + 1, sl]
                acc = acc + rows_v[buf, r0 + 2, sl]
                acc = acc + rows_v[buf, r0 + 3, sl]
                acc = acc + rows_v[buf, r0 + 4, sl]
                acc_v[buf, si, sl] = acc
            return 0

        lax.fori_loop(0, CS, seg, 0)

    start(gather_descs(0, 0, g0))

    def pair(k, _):
        ci0 = 2 * k
        ci1 = ci0 + 1
        start(gather_descs(ci1, 1, g1))
        wait(gather_descs(ci0, 0, g0))

        @pl.when(ci0 >= 2)
        def _():
            scatter_desc(ci0 - 2, 0, 0).wait()

        compute(0)
        scatter_desc(ci0, 0, 0).start()

        @pl.when(ci1 + 1 < nchunks)
        def _():
            start(gather_descs(ci1 + 1, 0, g0))

        wait(gather_descs(ci1, 1, g1))

        @pl.when(ci1 >= 2)
        def _():
            scatter_desc(ci1 - 2, 1, 1).wait()

        compute(1)
        scatter_desc(ci1, 1, 1).start()
        return 0

    lax.fori_loop(0, nchunks // 2, pair, 0)

    if nchunks % 2:
        ci = nchunks - 1
        wait(gather_descs(ci, 0, g0))
        scatter_desc(ci - 2, 0, 0).wait()
        compute(0)
        scatter_desc(ci, 0, 0).start()

    scatter_desc(nchunks - 1, 0, 0).wait()
    scatter_desc(nchunks - 2, 1, 1).wait()

    if has_extras:
        pltpu.sync_copy(eid_hbm.at[pl.ds(wid * R, R)],
                        ids_v.at[pl.ds(0, R)])
        descs = [
            pltpu.make_async_copy(
                g_hbm.at[ids_v.at[pl.ds(gi * SEG_G * L, SEG_G * L)]],
                rows_v.at[0, pl.ds(gi * SEG_G * L, SEG_G * L)], g0)
            for gi in range(GPC)
        ]
        start(descs)
        wait(descs)
        compute(0)
        pltpu.sync_copy(acc_v.at[0],
                        o_hbm.at[pl.ds(NW * n_per_w + wid * CS, CS)])


@functools.cache
def _sc_pool_side(n_per_w, n_extra):
    return pl.kernel(
        functools.partial(_sc_pool_side_body, n_per_w, n_extra > 0),
        mesh=plsc.VectorSubcoreMesh(core_axis_name="c", subcore_axis_name="s"),
        out_type=jax.ShapeDtypeStruct((NW * n_per_w + n_extra, H),
                                      jnp.float32),
        scratch_types=[
            pltpu.VMEM((n_per_w * L,), jnp.int32),
            pltpu.VMEM((2, R, H), jnp.float32),
            pltpu.VMEM((2, CS, H), jnp.float32),
            pltpu.SemaphoreType.DMA,
            pltpu.SemaphoreType.DMA,
            pltpu.SemaphoreType.DMA,
            pltpu.SemaphoreType.DMA,
        ],
        compiler_params=pltpu.CompilerParams(use_tc_tiling_on_sc=False),
    )


def _assemble_body(xnum_ref, vsum_ref, hsum_ref, xm_ref, nm_ref, cm_ref,
                   ex_ref, nbias_ref, wt_ref, emb_ref, bert_ref):
    eps = 1e-12
    bb = xnum_ref.shape[0]
    ex = ex_ref[...]
    nden = jnp.sum(nm_ref[...].astype(jnp.float32), axis=1, keepdims=True) + eps
    ncp_avg = ex[N_CAT:N_CAT + N_NUM] / nden
    cden = jnp.sum(cm_ref[...].astype(jnp.float32), axis=1, keepdims=True) + eps
    colp_avg = ex[:N_CAT] / cden
    bias_p = jnp.dot(nbias_ref[...], wt_ref[...],
                     preferred_element_type=jnp.float32)
    rden = 1.0 / (jnp.sum(xm_ref[...].astype(jnp.float32), axis=2) + eps)
    vsum = vsum_ref[...].reshape(bb, N_CAT, H)
    hsum = hsum_ref[...].reshape(bb, N_CAT, H)
    val_avg = vsum * rden[:, :, None]
    hdr_avg = hsum * rden[:, :, None]
    num_part = xnum_ref[...][:, :, None] * ncp_avg[None] + bias_p[None]
    cat_part = (colp_avg[None] + val_avg) * 0.5
    emb_ref[...] = jnp.concatenate([num_part, cat_part], axis=1)
    bert_ref[...] = hdr_avg


def _assemble(xnum, vsum, hsum_all, xm, nm, cm, nbias, wt):
    BB = 128
    nb = B // BB
    f = pl.pallas_call(
        _assemble_body,
        grid=(nb,),
        in_specs=[
            pl.BlockSpec((BB, N_NUM), lambda i: (i, 0)),
            pl.BlockSpec((BB * N_CAT, H), lambda i: (i, 0)),
            pl.BlockSpec((BB * N_CAT, H), lambda i: (i, 0)),
            pl.BlockSpec((BB, N_CAT, L), lambda i: (i, 0, 0)),
            pl.BlockSpec((N_NUM, L), lambda i: (0, 0)),
            pl.BlockSpec((N_CAT, L), lambda i: (0, 0)),
            pl.BlockSpec((64, H), lambda i: (SV // 64, 0)),
            pl.BlockSpec((1, D), lambda i: (0, 0)),
            pl.BlockSpec((D, H), lambda i: (0, 0)),
        ],
        out_specs=[
            pl.BlockSpec((BB, N_NUM + N_CAT, H), lambda i: (i, 0, 0)),
            pl.BlockSpec((BB, N_CAT, H), lambda i: (i, 0, 0)),
        ],
        out_shape=[
            jax.ShapeDtypeStruct((B, N_NUM + N_CAT, H), jnp.float32),
            jax.ShapeDtypeStruct((B, N_CAT, H), jnp.float32),
        ],
    )
    return f(xnum, vsum, hsum_all, xm, nm, cm, hsum_all, nbias, wt)


def kernel(x_num, num_col_input_ids, num_att_mask, x_cat_input_ids,
           x_cat_att_mask, col_cat_input_ids, col_cat_att_mask, header_table,
           value_table, norm_header_w, norm_header_b, norm_value_w,
           norm_value_b, num_bias, align_W):
    wt = align_W.T
    gv = _table_proj(value_table, norm_value_w, norm_value_b, wt, VV, VPV)
    gh = _table_proj(header_table, norm_header_w, norm_header_b, wt, VH, VPH)

    xm = x_cat_att_mask != 0
    spread = (jnp.arange(SV * L, dtype=jnp.int32) % NSPREAD).reshape(
        B, N_CAT, L)
    val_ids = jnp.where(xm, x_cat_input_ids, VV + spread).reshape(-1)
    hdr_ids = jnp.where(xm, x_cat_input_ids, VH + spread).reshape(-1)
    extra_ids = jnp.concatenate(
        [col_cat_input_ids.reshape(-1), num_col_input_ids.reshape(-1),
         jnp.zeros(((SE - N_CAT - N_NUM) * L,), jnp.int32)])
    extra_msk = jnp.concatenate(
        [col_cat_att_mask.reshape(-1), num_att_mask.reshape(-1),
         jnp.zeros(((SE - N_CAT - N_NUM) * L,), jnp.int32)])
    epos = jnp.arange(SE * L, dtype=jnp.int32)
    extra_idsm = jnp.where(extra_msk != 0, extra_ids, VH + epos % NSPREAD)

    vsum = _sc_pool_side(NV, 0)(gv, val_ids)
    hsum_all = _sc_pool_side(NV, SE)(gh, hdr_ids, extra_idsm)

    emb, bert = _assemble(x_num, vsum, hsum_all, x_cat_att_mask, num_att_mask,
                          col_cat_att_mask, num_bias.reshape(1, D), wt)
    return emb, bert


# speedup vs baseline: 1.0575x; 1.0008x over previous
import functools

import jax
import jax.numpy as jnp
from jax import lax
from jax.experimental import pallas as pl
from jax.experimental.pallas import tpu as pltpu
from jax.experimental.pallas import tpu_sc as plsc

B = 1024
N_NUM = 13
N_CAT = 26
L = 5
VH = 30522
VV = 100000
D = 768
H = 128

NC = 2
NS = 16
NW = NC * NS

CS = 64
SEG_G = 16
GPC = CS // SEG_G
R = CS * L
SV = B * N_CAT
SE = 2048
SH = SV + SE
NV = SV // NW
BLK = 2400
VPV = 100800
VPH = 31200
NSPREAD = 512


def _table_proj_body(tab_ref, wwt_ref, s2_ref, bwt_ref, o_ref, *, v_rows):
    pid = pl.program_id(0)
    x = tab_ref[...]
    mu = jnp.mean(x, axis=1, keepdims=True)
    msq = jnp.mean(x * x, axis=1, keepdims=True)
    rstd = lax.rsqrt(jnp.maximum(msq - mu * mu, 0.0) + 1e-5)
    p = jnp.dot(x, wwt_ref[...], preferred_element_type=jnp.float32)
    z = (p - mu * s2_ref[...]) * rstd + bwt_ref[...]
    rows = pid * BLK + lax.broadcasted_iota(jnp.int32, (BLK, 1), 0)
    o_ref[...] = jnp.where(rows < v_rows, z, 0.0)


def _table_proj(tab, w, b, wt, v_rows, vp_rows):
    nb = vp_rows // BLK
    in_cap = (v_rows + BLK - 1) // BLK - 1
    wwt = w[:, None] * wt
    s2 = jnp.sum(wwt, axis=0)[None]
    bwt = (b @ wt)[None]
    f = pl.pallas_call(
        functools.partial(_table_proj_body, v_rows=v_rows),
        grid=(nb,),
        in_specs=[
            pl.BlockSpec((BLK, D), lambda i: (jnp.minimum(i, in_cap), 0)),
            pl.BlockSpec((D, H), lambda i: (0, 0)),
            pl.BlockSpec((1, H), lambda i: (0, 0)),
            pl.BlockSpec((1, H), lambda i: (0, 0)),
        ],
        out_specs=pl.BlockSpec((BLK, H), lambda i: (i, 0)),
        out_shape=jax.ShapeDtypeStruct((vp_rows, H), jnp.float32),
    )
    return f(tab, wwt, s2, bwt)


def _sc_pool_side_body(n_per_w, has_extras, g_hbm, id_hbm, *rest):
    if has_extras:
        eid_hbm, o_hbm, ids_v, rows_v, acc_v, g0, g1, s0, s1 = rest
    else:
        o_hbm, ids_v, rows_v, acc_v, g0, g1, s0, s1 = rest
    wid = lax.axis_index("s") * NC + lax.axis_index("c")
    base = wid * n_per_w
    nw = n_per_w * L
    nchunks = n_per_w // CS
    pltpu.sync_copy(id_hbm.at[pl.ds(base * L, nw)], ids_v)

    def gather_descs(ci, buf, sem):
        return [
            pltpu.make_async_copy(
                g_hbm.at[ids_v.at[pl.ds(ci * R + gi * SEG_G * L, SEG_G * L)]],
                rows_v.at[buf, pl.ds(gi * SEG_G * L, SEG_G * L)], sem)
            for gi in range(GPC)
        ]

    def start(descs):
        for dsc in descs:
            dsc.start()

    def wait(descs):
        for dsc in descs:
            dsc.wait()

    def scatter_desc(ci, buf, sem):
        return pltpu.make_async_copy(
            acc_v.at[buf], o_hbm.at[pl.ds(base + ci * CS, CS)],
            s0 if sem == 0 else s1)

    def compute(buf):
        def seg(si, _):
            r0 = si * L
            for dc in range(H // 16):
                sl = pl.ds(dc * 16, 16)
                acc = rows_v[buf, r0, sl] + rows_v[buf, r0 + 1, sl]
                acc = acc + rows_v[buf, r0 + 2, sl]
                acc = acc + rows_v[buf, r0 + 3, sl]
                acc = acc + rows_v[buf, r0 + 4, sl]
                acc_v[buf, si, sl] = acc
            return 0

        lax.fori_loop(0, CS, seg, 0)

    start(gather_descs(0, 0, g0))

    def pair(k, _):
        ci0 = 2 * k
        ci1 = ci0 + 1
        start(gather_descs(ci1, 1, g1))
        wait(gather_descs(ci0, 0, g0))

        @pl.when(ci0 >= 2)
        def _():
            scatter_desc(ci0 - 2, 0, 0).wait()

        compute(0)
        scatter_desc(ci0, 0, 0).start()

        @pl.when(ci1 + 1 < nchunks)
        def _():
            start(gather_descs(ci1 + 1, 0, g0))

        wait(gather_descs(ci1, 1, g1))

        @pl.when(ci1 >= 2)
        def _():
            scatter_desc(ci1 - 2, 1, 1).wait()

        compute(1)
        scatter_desc(ci1, 1, 1).start()
        return 0

    lax.fori_loop(0, nchunks // 2, pair, 0)

    if nchunks % 2:
        ci = nchunks - 1
        wait(gather_descs(ci, 0, g0))
        scatter_desc(ci - 2, 0, 0).wait()
        compute(0)
        scatter_desc(ci, 0, 0).start()

    scatter_desc(nchunks - 1, 0, 0).wait()
    scatter_desc(nchunks - 2, 1, 1).wait()

    if has_extras:
        pltpu.sync_copy(eid_hbm.at[pl.ds(wid * R, R)],
                        ids_v.at[pl.ds(0, R)])
        descs = [
            pltpu.make_async_copy(
                g_hbm.at[ids_v.at[pl.ds(gi * SEG_G * L, SEG_G * L)]],
                rows_v.at[0, pl.ds(gi * SEG_G * L, SEG_G * L)], g0)
            for gi in range(GPC)
        ]
        start(descs)
        wait(descs)
        compute(0)
        pltpu.sync_copy(acc_v.at[0],
                        o_hbm.at[pl.ds(NW * n_per_w + wid * CS, CS)])


@functools.cache
def _sc_pool_side(n_per_w, n_extra):
    return pl.kernel(
        functools.partial(_sc_pool_side_body, n_per_w, n_extra > 0),
        mesh=plsc.VectorSubcoreMesh(core_axis_name="c", subcore_axis_name="s"),
        out_type=jax.ShapeDtypeStruct((NW * n_per_w + n_extra, H),
                                      jnp.float32),
        scratch_types=[
            pltpu.VMEM((n_per_w * L,), jnp.int32),
            pltpu.VMEM((2, R, H), jnp.float32),
            pltpu.VMEM((2, CS, H), jnp.float32),
            pltpu.SemaphoreType.DMA,
            pltpu.SemaphoreType.DMA,
            pltpu.SemaphoreType.DMA,
            pltpu.SemaphoreType.DMA,
        ],
        compiler_params=pltpu.CompilerParams(use_tc_tiling_on_sc=False),
    )


def _assemble_body(xnum_ref, vsum_ref, hsum_ref, xm_ref, nm_ref, cm_ref,
                   ex_ref, nbias_ref, wt_ref, emb_ref, bert_ref):
    eps = 1e-12
    bb = xnum_ref.shape[0]
    ex = ex_ref[...]
    nden = jnp.sum(nm_ref[...].astype(jnp.float32), axis=1, keepdims=True) + eps
    ncp_avg = ex[N_CAT:N_CAT + N_NUM] / nden
    cden = jnp.sum(cm_ref[...].astype(jnp.float32), axis=1, keepdims=True) + eps
    colp_avg = ex[:N_CAT] / cden
    bias_p = jnp.dot(nbias_ref[...], wt_ref[...],
                     preferred_element_type=jnp.float32)
    rden = 1.0 / (jnp.sum(xm_ref[...].astype(jnp.float32), axis=2) + eps)
    vsum = vsum_ref[...].reshape(bb, N_CAT, H)
    hsum = hsum_ref[...].reshape(bb, N_CAT, H)
    val_avg = vsum * rden[:, :, None]
    hdr_avg = hsum * rden[:, :, None]
    num_part = xnum_ref[...][:, :, None] * ncp_avg[None] + bias_p[None]
    cat_part = (colp_avg[None] + val_avg) * 0.5
    emb_ref[...] = jnp.concatenate([num_part, cat_part], axis=1)
    bert_ref[...] = hdr_avg


def _assemble(xnum, vsum, hsum_all, xm, nm, cm, nbias, wt):
    BB = 128
    nb = B // BB
    f = pl.pallas_call(
        _assemble_body,
        grid=(nb,),
        in_specs=[
            pl.BlockSpec((BB, N_NUM), lambda i: (i, 0)),
            pl.BlockSpec((BB * N_CAT, H), lambda i: (i, 0)),
            pl.BlockSpec((BB * N_CAT, H), lambda i: (i, 0)),
            pl.BlockSpec((BB, N_CAT, L), lambda i: (i, 0, 0)),
            pl.BlockSpec((N_NUM, L), lambda i: (0, 0)),
            pl.BlockSpec((N_CAT, L), lambda i: (0, 0)),
            pl.BlockSpec((64, H), lambda i: (SV // 64, 0)),
            pl.BlockSpec((1, D), lambda i: (0, 0)),
            pl.BlockSpec((D, H), lambda i: (0, 0)),
        ],
        out_specs=[
            pl.BlockSpec((BB, N_NUM + N_CAT, H), lambda i: (i, 0, 0)),
            pl.BlockSpec((BB, N_CAT, H), lambda i: (i, 0, 0)),
        ],
        out_shape=[
            jax.ShapeDtypeStruct((B, N_NUM + N_CAT, H), jnp.float32),
            jax.ShapeDtypeStruct((B, N_CAT, H), jnp.float32),
        ],
    )
    return f(xnum, vsum, hsum_all, xm, nm, cm, hsum_all, nbias, wt)


def kernel(x_num, num_col_input_ids, num_att_mask, x_cat_input_ids,
           x_cat_att_mask, col_cat_input_ids, col_cat_att_mask, header_table,
           value_table, norm_header_w, norm_header_b, norm_value_w,
           norm_value_b, num_bias, align_W):
    wt = align_W.T
    gv = _table_proj(value_table, norm_value_w, norm_value_b, wt, VV, VPV)
    gh = _table_proj(header_table, norm_header_w, norm_header_b, wt, VH, VPH)

    xm = x_cat_att_mask != 0
    spread = (jnp.arange(SV * L, dtype=jnp.int32) % NSPREAD).reshape(
        B, N_CAT, L)
    val_ids = jnp.where(xm, x_cat_input_ids, VV + spread).reshape(-1)
    hdr_ids = jnp.where(xm, x_cat_input_ids, VH + spread).reshape(-1)
    extra_ids = jnp.concatenate(
        [col_cat_input_ids.reshape(-1), num_col_input_ids.reshape(-1),
         jnp.zeros(((SE - N_CAT - N_NUM) * L,), jnp.int32)])
    extra_msk = jnp.concatenate(
        [col_cat_att_mask.reshape(-1), num_att_mask.reshape(-1),
         jnp.zeros(((SE - N_CAT - N_NUM) * L,), jnp.int32)])
    epos = jnp.arange(SE * L, dtype=jnp.int32)
    extra_idsm = jnp.where(extra_msk != 0, extra_ids, VH + epos % NSPREAD)

    vsum = _sc_pool_side(NV, 0)(gv, val_ids)
    hsum_all = _sc_pool_side(NV, SE)(gh, hdr_ids, extra_idsm)

    emb, bert = _assemble(x_num, vsum, hsum_all, x_cat_att_mask, num_att_mask,
                          col_cat_att_mask, num_bias.reshape(1, D), wt)
    return emb, bert
